# baseline (speedup 1.0000x reference)
import functools

import jax
import jax.numpy as jnp
from jax import lax
from jax.experimental import pallas as pl
from jax.experimental.pallas import tpu as pltpu
from jax.experimental.pallas import tpu_sc as plsc

N_ = 10000
D_ = 128
H1 = 8
HC1 = 64
OUT_ = 16
NPAD = 10112
NCORE = 2
NTILE = 16
CHUNK = 128
NCHUNK = 82
EPAD = NCORE * NTILE * NCHUNK * CHUNK
ROWS_PT = NPAD // NTILE
RB = NPAD // 8

_mesh = plsc.VectorSubcoreMesh(core_axis_name="c", subcore_axis_name="s")
_sc_params = pltpu.CompilerParams(use_tc_tiling_on_sc=False,
                                  needs_layout_passes=False)


def _dg16(v, idx):
    return lax.gather(
        v, idx[:, None],
        lax.GatherDimensionNumbers(
            offset_dims=(), collapsed_slice_dims=(0,), start_index_map=(0,)),
        slice_sizes=(1,),
        mode=lax.GatherScatterMode.PROMISE_IN_BOUNDS)


def _ka_body(x_ref, w1_ref, ms_ref, md_ref, h1_ref, a1a_ref, a1b_ref):
    h = jnp.dot(x_ref[...], w1_ref[...], preferred_element_type=jnp.float32)
    h1_ref[...] = h
    asrc = jnp.dot(h, ms_ref[...], preferred_element_type=jnp.float32)
    adst = jnp.dot(h, md_ref[...], preferred_element_type=jnp.float32)
    a1a_ref[...] = jnp.concatenate([asrc, adst], axis=1)
    a1b_ref[...] = jnp.concatenate([adst, asrc], axis=1)


def _run_ka(xp, W1, msrc, mdst):
    blk = lambda r, c: pl.BlockSpec((r, c), lambda i: (i, 0))
    rep = lambda r, c: pl.BlockSpec((r, c), lambda i: (0, 0))
    return pl.pallas_call(
        _ka_body,
        grid=(NPAD // RB,),
        in_specs=[blk(RB, D_), rep(D_, HC1), rep(HC1, H1), rep(HC1, H1)],
        out_specs=[blk(RB, HC1), blk(RB, 16), blk(RB, 16)],
        out_shape=[
            jax.ShapeDtypeStruct((NPAD, HC1), jnp.float32),
            jax.ShapeDtypeStruct((NPAD, 16), jnp.float32),
            jax.ShapeDtypeStruct((NPAD, 16), jnp.float32),
        ],
    )(xp, W1, msrc, mdst)


def _edge1_body(h1_hbm, a1a_hbm, a1b_hbm, src_hbm, dst_hbm,
                acc_out, s_out,
                src_v, dst_v, S_v, D_v, h_v, w_v, msg_v, acc_sh, s_sh,
                sem, sem2):
    c = lax.axis_index("c")
    s = lax.axis_index("s")
    wid = c * NTILE + s
    z16 = jnp.zeros((16,), jnp.float32)

    def zbody(i, carry):
        w_v[0, i] = z16
        for kk in range(4):
            msg_v[0, i, pl.ds(16 * kk, 16)] = z16
        return carry
    lax.fori_loop(0, CHUNK, zbody, 0)

    base = s * ROWS_PT
    for j in range(4):
        pltpu.sync_copy(msg_v.at[0], acc_sh.at[pl.ds(base + j * CHUNK, CHUNK)])
        pltpu.sync_copy(w_v.at[0], s_sh.at[pl.ds(base + j * CHUNK, CHUNK)])
    tail = ROWS_PT - 4 * CHUNK
    pltpu.sync_copy(msg_v.at[0].at[pl.ds(0, tail)],
                    acc_sh.at[pl.ds(base + 4 * CHUNK, tail)])
    pltpu.sync_copy(w_v.at[0].at[pl.ds(0, tail)],
                    s_sh.at[pl.ds(base + 4 * CHUNK, tail)])
    plsc.subcore_barrier()

    iota16 = lax.iota(jnp.int32, 16)
    hi8 = lax.shift_right_logical(iota16, 3)
    pats = [2 * kk + hi8 for kk in range(4)]

    pltpu.sync_copy(src_hbm.at[pl.ds(wid * NCHUNK, NCHUNK)], src_v)
    pltpu.sync_copy(dst_hbm.at[pl.ds(wid * NCHUNK, NCHUNK)], dst_v)

    def gathers(k, b):
        return [
            pltpu.make_async_copy(a1a_hbm.at[src_v.at[k]], S_v.at[b], sem),
            pltpu.make_async_copy(a1b_hbm.at[dst_v.at[k]], D_v.at[b], sem),
            pltpu.make_async_copy(h1_hbm.at[src_v.at[k]], h_v.at[b], sem),
        ]

    def scatters(k, b):
        return [
            pltpu.make_async_copy(w_v.at[b], s_sh.at[dst_v.at[k]], sem2),
            pltpu.make_async_copy(msg_v.at[b], acc_sh.at[dst_v.at[k]], sem2),
        ]

    for b in range(2):
        for g in gathers(b, b):
            g.start()

    def cbody(j, carry):
        for b in range(2):
            k = 2 * j + b
            for g in gathers(k, b):
                g.wait()

            @pl.when(k >= 2)
            def _():
                for sc in scatters(k - 2, b):
                    sc.wait()

            @plsc.parallel_loop(0, CHUNK, 1, unroll=8)
            def _(i):
                t = S_v[b, i] + D_v[b, i]
                w = jnp.exp(jnp.maximum(t, t * 0.2))
                w_v[b, i] = w
                for kk in range(4):
                    wexp = _dg16(w, pats[kk])
                    msg_v[b, i, pl.ds(16 * kk, 16)] = (
                        h_v[b, i, pl.ds(16 * kk, 16)] * wexp)

            pltpu.async_copy(w_v.at[b], s_sh.at[dst_v.at[k]], sem2, add=True)
            pltpu.async_copy(msg_v.at[b], acc_sh.at[dst_v.at[k]], sem2,
                             add=True)

            @pl.when(k + 2 < NCHUNK)
            def _():
                for g in gathers(k + 2, b):
                    g.start()
        return carry
    lax.fori_loop(0, NCHUNK // 2, cbody, 0)

    for b in range(2):
        for sc in scatters(NCHUNK - 2 + b, b):
            sc.wait()

    plsc.subcore_barrier()
    pltpu.sync_copy(acc_sh.at[pl.ds(base, ROWS_PT)],
                    acc_out.at[pl.ds(c * NPAD + base, ROWS_PT)])
    pltpu.sync_copy(s_sh.at[pl.ds(base, ROWS_PT)],
                    s_out.at[pl.ds(c * NPAD + base, ROWS_PT)])


_edge1 = functools.partial(
    pl.kernel,
    out_type=[
        jax.ShapeDtypeStruct((NCORE * NPAD, HC1), jnp.float32),
        jax.ShapeDtypeStruct((NCORE * NPAD, 16), jnp.float32),
    ],
    mesh=_mesh,
    scratch_types=[
        pltpu.VMEM((NCHUNK, CHUNK), jnp.int32),
        pltpu.VMEM((NCHUNK, CHUNK), jnp.int32),
        pltpu.VMEM((2, CHUNK, 16), jnp.float32),
        pltpu.VMEM((2, CHUNK, 16), jnp.float32),
        pltpu.VMEM((2, CHUNK, HC1), jnp.float32),
        pltpu.VMEM((2, CHUNK, 16), jnp.float32),
        pltpu.VMEM((2, CHUNK, HC1), jnp.float32),
        pltpu.VMEM_SHARED((NPAD, HC1), jnp.float32),
        pltpu.VMEM_SHARED((NPAD, 16), jnp.float32),
        pltpu.SemaphoreType.DMA,
        pltpu.SemaphoreType.DMA,
    ],
    compiler_params=_sc_params,
)(_edge1_body)


def _kc_body(a0_ref, a1_ref, s0_ref, s1_ref, b1_ref, w2_ref, erep_ref,
             as2_ref, ad2_ref, h2_ref, a2a_ref, a2b_ref):
    acc = a0_ref[...] + a1_ref[...]
    ssum = s0_ref[...][:, :H1] + s1_ref[...][:, :H1]
    inv = 1.0 / (ssum + 1e-16)
    inv_rep = jnp.dot(inv, erep_ref[...], preferred_element_type=jnp.float32)
    o1 = acc * inv_rep + b1_ref[...]
    hh = jnp.where(o1 > 0, o1, jnp.exp(o1) - 1.0)
    h2 = jnp.dot(hh, w2_ref[...], preferred_element_type=jnp.float32)
    h2_ref[...] = h2
    es = jnp.dot(h2, as2_ref[...], preferred_element_type=jnp.float32)
    ed = jnp.dot(h2, ad2_ref[...], preferred_element_type=jnp.float32)
    a2a_ref[...] = jnp.broadcast_to(es, (es.shape[0], 16))
    a2b_ref[...] = jnp.broadcast_to(ed, (ed.shape[0], 16))


def _run_kc(a0, a1, s0, s1, b1row, W2, erep, as2, ad2):
    blk = lambda r, c: pl.BlockSpec((r, c), lambda i: (i, 0))
    rep = lambda r, c: pl.BlockSpec((r, c), lambda i: (0, 0))
    return pl.pallas_call(
        _kc_body,
        grid=(NPAD // RB,),
        in_specs=[blk(RB, HC1), blk(RB, HC1), blk(RB, 16), blk(RB, 16),
                  rep(1, HC1), rep(HC1, OUT_), rep(H1, HC1),
                  rep(OUT_, 1), rep(OUT_, 1)],
        out_specs=[blk(RB, OUT_), blk(RB, 16), blk(RB, 16)],
        out_shape=[
            jax.ShapeDtypeStruct((NPAD, OUT_), jnp.float32),
            jax.ShapeDtypeStruct((NPAD, 16), jnp.float32),
            jax.ShapeDtypeStruct((NPAD, 16), jnp.float32),
        ],
    )(a0, a1, s0, s1, b1row, W2, erep, as2, ad2)


def _edge2_body(h2_hbm, a2a_hbm, a2b_hbm, src_hbm, dst_hbm,
                acc_out, s_out,
                src_v, dst_v, S_v, D_v, h_v, w_v, msg_v, acc_sh, s_sh,
                sem, sem2):
    c = lax.axis_index("c")
    s = lax.axis_index("s")
    wid = c * NTILE + s
    z16 = jnp.zeros((16,), jnp.float32)

    def zbody(i, carry):
        w_v[0, i] = z16
        msg_v[0, i] = z16
        return carry
    lax.fori_loop(0, CHUNK, zbody, 0)

    base = s * ROWS_PT
    for j in range(4):
        pltpu.sync_copy(msg_v.at[0], acc_sh.at[pl.ds(base + j * CHUNK, CHUNK)])
        pltpu.sync_copy(w_v.at[0], s_sh.at[pl.ds(base + j * CHUNK, CHUNK)])
    tail = ROWS_PT - 4 * CHUNK
    pltpu.sync_copy(msg_v.at[0].at[pl.ds(0, tail)],
                    acc_sh.at[pl.ds(base + 4 * CHUNK, tail)])
    pltpu.sync_copy(w_v.at[0].at[pl.ds(0, tail)],
                    s_sh.at[pl.ds(base + 4 * CHUNK, tail)])
    plsc.subcore_barrier()

    pltpu.sync_copy(src_hbm.at[pl.ds(wid * NCHUNK, NCHUNK)], src_v)
    pltpu.sync_copy(dst_hbm.at[pl.ds(wid * NCHUNK, NCHUNK)], dst_v)

    def gathers(k, b):
        return [
            pltpu.make_async_copy(a2a_hbm.at[src_v.at[k]], S_v.at[b], sem),
            pltpu.make_async_copy(a2b_hbm.at[dst_v.at[k]], D_v.at[b], sem),
            pltpu.make_async_copy(h2_hbm.at[src_v.at[k]], h_v.at[b], sem),
        ]

    def scatters(k, b):
        return [
            pltpu.make_async_copy(w_v.at[b], s_sh.at[dst_v.at[k]], sem2),
            pltpu.make_async_copy(msg_v.at[b], acc_sh.at[dst_v.at[k]], sem2),
        ]

    for b in range(2):
        for g in gathers(b, b):
            g.start()

    def cbody(j, carry):
        for b in range(2):
            k = 2 * j + b
            for g in gathers(k, b):
                g.wait()

            @pl.when(k >= 2)
            def _():
                for sc in scatters(k - 2, b):
                    sc.wait()

            @plsc.parallel_loop(0, CHUNK, 1, unroll=8)
            def _(i):
                t = S_v[b, i] + D_v[b, i]
                w = jnp.exp(jnp.maximum(t, t * 0.2))
                w_v[b, i] = w
                msg_v[b, i] = h_v[b, i] * w

            pltpu.async_copy(w_v.at[b], s_sh.at[dst_v.at[k]], sem2, add=True)
            pltpu.async_copy(msg_v.at[b], acc_sh.at[dst_v.at[k]], sem2,
                             add=True)

            @pl.when(k + 2 < NCHUNK)
            def _():
                for g in gathers(k + 2, b):
                    g.start()
        return carry
    lax.fori_loop(0, NCHUNK // 2, cbody, 0)

    for b in range(2):
        for sc in scatters(NCHUNK - 2 + b, b):
            sc.wait()

    plsc.subcore_barrier()
    pltpu.sync_copy(acc_sh.at[pl.ds(base, ROWS_PT)],
                    acc_out.at[pl.ds(c * NPAD + base, ROWS_PT)])
    pltpu.sync_copy(s_sh.at[pl.ds(base, ROWS_PT)],
                    s_out.at[pl.ds(c * NPAD + base, ROWS_PT)])


_edge2 = functools.partial(
    pl.kernel,
    out_type=[
        jax.ShapeDtypeStruct((NCORE * NPAD, OUT_), jnp.float32),
        jax.ShapeDtypeStruct((NCORE * NPAD, 16), jnp.float32),
    ],
    mesh=_mesh,
    scratch_types=[
        pltpu.VMEM((NCHUNK, CHUNK), jnp.int32),
        pltpu.VMEM((NCHUNK, CHUNK), jnp.int32),
        pltpu.VMEM((2, CHUNK, 16), jnp.float32),
        pltpu.VMEM((2, CHUNK, 16), jnp.float32),
        pltpu.VMEM((2, CHUNK, OUT_), jnp.float32),
        pltpu.VMEM((2, CHUNK, 16), jnp.float32),
        pltpu.VMEM((2, CHUNK, OUT_), jnp.float32),
        pltpu.VMEM_SHARED((NPAD, OUT_), jnp.float32),
        pltpu.VMEM_SHARED((NPAD, 16), jnp.float32),
        pltpu.SemaphoreType.DMA,
        pltpu.SemaphoreType.DMA,
    ],
    compiler_params=_sc_params,
)(_edge2_body)


def _ke_body(a0_ref, a1_ref, s0_ref, s1_ref, b2_ref, out_ref):
    acc = a0_ref[...] + a1_ref[...]
    ssum = s0_ref[...][:, 0:1] + s1_ref[...][:, 0:1]
    out_ref[...] = acc / (ssum + 1e-16) + b2_ref[...]


def _run_ke(a0, a1, s0, s1, b2row):
    blk = lambda r, c: pl.BlockSpec((r, c), lambda i: (i, 0))
    rep = lambda r, c: pl.BlockSpec((r, c), lambda i: (0, 0))
    return pl.pallas_call(
        _ke_body,
        grid=(5,),
        in_specs=[blk(2000, OUT_), blk(2000, OUT_), blk(2000, 16),
                  blk(2000, 16), rep(1, OUT_)],
        out_specs=blk(2000, OUT_),
        out_shape=jax.ShapeDtypeStruct((N_, OUT_), jnp.float32),
    )(a0, a1, s0, s1, b2row)


def kernel(x, edge_index, W1, a_src1, a_dst1, b1, W2, a_src2, a_dst2, b2):
    E = edge_index.shape[1]
    npad_e = EPAD - E - N_
    loop = jnp.arange(N_, dtype=jnp.int32)
    padidx = jnp.full((npad_e,), N_, dtype=jnp.int32)
    src = jnp.concatenate([edge_index[0], loop, padidx]).reshape(
        EPAD // CHUNK, CHUNK)
    dst = jnp.concatenate([edge_index[1], loop, padidx]).reshape(
        EPAD // CHUNK, CHUNK)

    xp = jnp.pad(x, ((0, NPAD - N_), (0, 0)))
    rows = jnp.arange(HC1)
    msrc = jnp.zeros((HC1, H1), jnp.float32).at[rows, rows // 8].set(
        a_src1.reshape(HC1))
    mdst = jnp.zeros((HC1, H1), jnp.float32).at[rows, rows // 8].set(
        a_dst1.reshape(HC1))
    erep = jnp.zeros((H1, HC1), jnp.float32).at[rows // 8, rows].set(1.0)

    h1, a1a, a1b = _run_ka(xp, W1, msrc, mdst)
    acc1, s1 = _edge1(h1, a1a, a1b, src, dst)
    h2, a2a, a2b = _run_kc(acc1[:NPAD], acc1[NPAD:], s1[:NPAD], s1[NPAD:],
                           b1.reshape(1, HC1), W2, erep,
                           a_src2.reshape(OUT_, 1), a_dst2.reshape(OUT_, 1))
    acc2, s2 = _edge2(h2, a2a, a2b, src, dst)
    return _run_ke(acc2[:N_], acc2[NPAD:NPAD + N_],
                   s2[:N_], s2[NPAD:NPAD + N_], b2.reshape(1, OUT_))


# speedup vs baseline: 1.0053x; 1.0053x over previous
import functools

import jax
import jax.numpy as jnp
from jax import lax
from jax.experimental import pallas as pl
from jax.experimental.pallas import tpu as pltpu
from jax.experimental.pallas import tpu_sc as plsc

N_ = 10000
D_ = 128
H1 = 8
HC1 = 64
OUT_ = 16
NPAD = 10112
NCORE = 2
NTILE = 16
CHUNK = 128
NCHUNK = 82
EPAD = NCORE * NTILE * NCHUNK * CHUNK
ROWS_PT = NPAD // NTILE
RB = NPAD // 8

_mesh = plsc.VectorSubcoreMesh(core_axis_name="c", subcore_axis_name="s")
_sc_params = pltpu.CompilerParams(use_tc_tiling_on_sc=False,
                                  needs_layout_passes=False)


def _dg16(v, idx):
    return lax.gather(
        v, idx[:, None],
        lax.GatherDimensionNumbers(
            offset_dims=(), collapsed_slice_dims=(0,), start_index_map=(0,)),
        slice_sizes=(1,),
        mode=lax.GatherScatterMode.PROMISE_IN_BOUNDS)


def _ka_body(x_ref, w1_ref, ms_ref, md_ref, h1_ref, a1a_ref, a1b_ref):
    h = jnp.dot(x_ref[...], w1_ref[...], preferred_element_type=jnp.float32)
    h1_ref[...] = h
    asrc = jnp.dot(h, ms_ref[...], preferred_element_type=jnp.float32)
    adst = jnp.dot(h, md_ref[...], preferred_element_type=jnp.float32)
    a1a_ref[...] = jnp.concatenate([asrc, adst], axis=1)
    a1b_ref[...] = jnp.concatenate([adst, asrc], axis=1)


def _run_ka(xp, W1, msrc, mdst):
    blk = lambda r, c: pl.BlockSpec((r, c), lambda i: (i, 0))
    rep = lambda r, c: pl.BlockSpec((r, c), lambda i: (0, 0))
    return pl.pallas_call(
        _ka_body,
        grid=(NPAD // RB,),
        in_specs=[blk(RB, D_), rep(D_, HC1), rep(HC1, H1), rep(HC1, H1)],
        out_specs=[blk(RB, HC1), blk(RB, 16), blk(RB, 16)],
        out_shape=[
            jax.ShapeDtypeStruct((NPAD, HC1), jnp.float32),
            jax.ShapeDtypeStruct((NPAD, 16), jnp.float32),
            jax.ShapeDtypeStruct((NPAD, 16), jnp.float32),
        ],
    )(xp, W1, msrc, mdst)


def _edge1_body(h1_hbm, a1a_hbm, a1b_hbm, src_hbm, dst_hbm,
                acc_out, s_out,
                src_v, dst_v, S_v, D_v, h_v, w_v, msg_v, acc_sh, s_sh,
                sem, sem2):
    c = lax.axis_index("c")
    s = lax.axis_index("s")
    wid = c * NTILE + s
    z16 = jnp.zeros((16,), jnp.float32)

    def zbody(i, carry):
        w_v[0, i] = z16
        for kk in range(4):
            msg_v[0, i, pl.ds(16 * kk, 16)] = z16
        return carry
    lax.fori_loop(0, CHUNK, zbody, 0)

    base = s * ROWS_PT
    for j in range(4):
        pltpu.sync_copy(msg_v.at[0], acc_sh.at[pl.ds(base + j * CHUNK, CHUNK)])
        pltpu.sync_copy(w_v.at[0], s_sh.at[pl.ds(base + j * CHUNK, CHUNK)])
    tail = ROWS_PT - 4 * CHUNK
    pltpu.sync_copy(msg_v.at[0].at[pl.ds(0, tail)],
                    acc_sh.at[pl.ds(base + 4 * CHUNK, tail)])
    pltpu.sync_copy(w_v.at[0].at[pl.ds(0, tail)],
                    s_sh.at[pl.ds(base + 4 * CHUNK, tail)])
    plsc.subcore_barrier()

    iota16 = lax.iota(jnp.int32, 16)
    hi8 = lax.shift_right_logical(iota16, 3)
    pats = [2 * kk + hi8 for kk in range(4)]

    pltpu.sync_copy(src_hbm.at[pl.ds(wid * NCHUNK, NCHUNK)], src_v)
    pltpu.sync_copy(dst_hbm.at[pl.ds(wid * NCHUNK, NCHUNK)], dst_v)

    def gathers(k, b):
        return [
            pltpu.make_async_copy(a1a_hbm.at[src_v.at[k]], S_v.at[b], sem),
            pltpu.make_async_copy(a1b_hbm.at[dst_v.at[k]], D_v.at[b], sem),
            pltpu.make_async_copy(h1_hbm.at[src_v.at[k]], h_v.at[b], sem),
        ]

    def scatters(k, b):
        return [
            pltpu.make_async_copy(w_v.at[b], s_sh.at[dst_v.at[k]], sem2),
            pltpu.make_async_copy(msg_v.at[b], acc_sh.at[dst_v.at[k]], sem2),
        ]

    for b in range(2):
        for g in gathers(b, b):
            g.start()

    def cbody(j, carry):
        for b in range(2):
            k = 2 * j + b
            for g in gathers(k, b):
                g.wait()

            @pl.when(k >= 2)
            def _():
                for sc in scatters(k - 2, b):
                    sc.wait()

            @plsc.parallel_loop(0, CHUNK, 1, unroll=4)
            def _(i):
                t = S_v[b, i] + D_v[b, i]
                w = jnp.exp(jnp.maximum(t, t * 0.2))
                w_v[b, i] = w
                for kk in range(4):
                    wexp = _dg16(w, pats[kk])
                    msg_v[b, i, pl.ds(16 * kk, 16)] = (
                        h_v[b, i, pl.ds(16 * kk, 16)] * wexp)

            pltpu.async_copy(w_v.at[b], s_sh.at[dst_v.at[k]], sem2, add=True)
            pltpu.async_copy(msg_v.at[b], acc_sh.at[dst_v.at[k]], sem2,
                             add=True)

            @pl.when(k + 2 < NCHUNK)
            def _():
                for g in gathers(k + 2, b):
                    g.start()
        return carry
    lax.fori_loop(0, NCHUNK // 2, cbody, 0)

    for b in range(2):
        for sc in scatters(NCHUNK - 2 + b, b):
            sc.wait()

    plsc.subcore_barrier()
    pltpu.sync_copy(acc_sh.at[pl.ds(base, ROWS_PT)],
                    acc_out.at[pl.ds(c * NPAD + base, ROWS_PT)])
    pltpu.sync_copy(s_sh.at[pl.ds(base, ROWS_PT)],
                    s_out.at[pl.ds(c * NPAD + base, ROWS_PT)])


_edge1 = functools.partial(
    pl.kernel,
    out_type=[
        jax.ShapeDtypeStruct((NCORE * NPAD, HC1), jnp.float32),
        jax.ShapeDtypeStruct((NCORE * NPAD, 16), jnp.float32),
    ],
    mesh=_mesh,
    scratch_types=[
        pltpu.VMEM((NCHUNK, CHUNK), jnp.int32),
        pltpu.VMEM((NCHUNK, CHUNK), jnp.int32),
        pltpu.VMEM((2, CHUNK, 16), jnp.float32),
        pltpu.VMEM((2, CHUNK, 16), jnp.float32),
        pltpu.VMEM((2, CHUNK, HC1), jnp.float32),
        pltpu.VMEM((2, CHUNK, 16), jnp.float32),
        pltpu.VMEM((2, CHUNK, HC1), jnp.float32),
        pltpu.VMEM_SHARED((NPAD, HC1), jnp.float32),
        pltpu.VMEM_SHARED((NPAD, 16), jnp.float32),
        pltpu.SemaphoreType.DMA,
        pltpu.SemaphoreType.DMA,
    ],
    compiler_params=_sc_params,
)(_edge1_body)


def _kc_body(a0_ref, a1_ref, s0_ref, s1_ref, b1_ref, w2_ref, erep_ref,
             as2_ref, ad2_ref, h2_ref, a2a_ref, a2b_ref):
    acc = a0_ref[...] + a1_ref[...]
    ssum = s0_ref[...][:, :H1] + s1_ref[...][:, :H1]
    inv = 1.0 / (ssum + 1e-16)
    inv_rep = jnp.dot(inv, erep_ref[...], preferred_element_type=jnp.float32)
    o1 = acc * inv_rep + b1_ref[...]
    hh = jnp.where(o1 > 0, o1, jnp.exp(o1) - 1.0)
    h2 = jnp.dot(hh, w2_ref[...], preferred_element_type=jnp.float32)
    h2_ref[...] = h2
    es = jnp.dot(h2, as2_ref[...], preferred_element_type=jnp.float32)
    ed = jnp.dot(h2, ad2_ref[...], preferred_element_type=jnp.float32)
    a2a_ref[...] = jnp.broadcast_to(es, (es.shape[0], 16))
    a2b_ref[...] = jnp.broadcast_to(ed, (ed.shape[0], 16))


def _run_kc(a0, a1, s0, s1, b1row, W2, erep, as2, ad2):
    blk = lambda r, c: pl.BlockSpec((r, c), lambda i: (i, 0))
    rep = lambda r, c: pl.BlockSpec((r, c), lambda i: (0, 0))
    return pl.pallas_call(
        _kc_body,
        grid=(NPAD // RB,),
        in_specs=[blk(RB, HC1), blk(RB, HC1), blk(RB, 16), blk(RB, 16),
                  rep(1, HC1), rep(HC1, OUT_), rep(H1, HC1),
                  rep(OUT_, 1), rep(OUT_, 1)],
        out_specs=[blk(RB, OUT_), blk(RB, 16), blk(RB, 16)],
        out_shape=[
            jax.ShapeDtypeStruct((NPAD, OUT_), jnp.float32),
            jax.ShapeDtypeStruct((NPAD, 16), jnp.float32),
            jax.ShapeDtypeStruct((NPAD, 16), jnp.float32),
        ],
    )(a0, a1, s0, s1, b1row, W2, erep, as2, ad2)


def _edge2_body(h2_hbm, a2a_hbm, a2b_hbm, src_hbm, dst_hbm,
                acc_out, s_out,
                src_v, dst_v, S_v, D_v, h_v, w_v, msg_v, acc_sh, s_sh,
                sem, sem2):
    c = lax.axis_index("c")
    s = lax.axis_index("s")
    wid = c * NTILE + s
    z16 = jnp.zeros((16,), jnp.float32)

    def zbody(i, carry):
        w_v[0, i] = z16
        msg_v[0, i] = z16
        return carry
    lax.fori_loop(0, CHUNK, zbody, 0)

    base = s * ROWS_PT
    for j in range(4):
        pltpu.sync_copy(msg_v.at[0], acc_sh.at[pl.ds(base + j * CHUNK, CHUNK)])
        pltpu.sync_copy(w_v.at[0], s_sh.at[pl.ds(base + j * CHUNK, CHUNK)])
    tail = ROWS_PT - 4 * CHUNK
    pltpu.sync_copy(msg_v.at[0].at[pl.ds(0, tail)],
                    acc_sh.at[pl.ds(base + 4 * CHUNK, tail)])
    pltpu.sync_copy(w_v.at[0].at[pl.ds(0, tail)],
                    s_sh.at[pl.ds(base + 4 * CHUNK, tail)])
    plsc.subcore_barrier()

    pltpu.sync_copy(src_hbm.at[pl.ds(wid * NCHUNK, NCHUNK)], src_v)
    pltpu.sync_copy(dst_hbm.at[pl.ds(wid * NCHUNK, NCHUNK)], dst_v)

    def gathers(k, b):
        return [
            pltpu.make_async_copy(a2a_hbm.at[src_v.at[k]], S_v.at[b], sem),
            pltpu.make_async_copy(a2b_hbm.at[dst_v.at[k]], D_v.at[b], sem),
            pltpu.make_async_copy(h2_hbm.at[src_v.at[k]], h_v.at[b], sem),
        ]

    def scatters(k, b):
        return [
            pltpu.make_async_copy(w_v.at[b], s_sh.at[dst_v.at[k]], sem2),
            pltpu.make_async_copy(msg_v.at[b], acc_sh.at[dst_v.at[k]], sem2),
        ]

    for b in range(2):
        for g in gathers(b, b):
            g.start()

    def cbody(j, carry):
        for b in range(2):
            k = 2 * j + b
            for g in gathers(k, b):
                g.wait()

            @pl.when(k >= 2)
            def _():
                for sc in scatters(k - 2, b):
                    sc.wait()

            @plsc.parallel_loop(0, CHUNK, 1, unroll=8)
            def _(i):
                t = S_v[b, i] + D_v[b, i]
                w = jnp.exp(jnp.maximum(t, t * 0.2))
                w_v[b, i] = w
                msg_v[b, i] = h_v[b, i] * w

            pltpu.async_copy(w_v.at[b], s_sh.at[dst_v.at[k]], sem2, add=True)
            pltpu.async_copy(msg_v.at[b], acc_sh.at[dst_v.at[k]], sem2,
                             add=True)

            @pl.when(k + 2 < NCHUNK)
            def _():
                for g in gathers(k + 2, b):
                    g.start()
        return carry
    lax.fori_loop(0, NCHUNK // 2, cbody, 0)

    for b in range(2):
        for sc in scatters(NCHUNK - 2 + b, b):
            sc.wait()

    plsc.subcore_barrier()
    pltpu.sync_copy(acc_sh.at[pl.ds(base, ROWS_PT)],
                    acc_out.at[pl.ds(c * NPAD + base, ROWS_PT)])
    pltpu.sync_copy(s_sh.at[pl.ds(base, ROWS_PT)],
                    s_out.at[pl.ds(c * NPAD + base, ROWS_PT)])


_edge2 = functools.partial(
    pl.kernel,
    out_type=[
        jax.ShapeDtypeStruct((NCORE * NPAD, OUT_), jnp.float32),
        jax.ShapeDtypeStruct((NCORE * NPAD, 16), jnp.float32),
    ],
    mesh=_mesh,
    scratch_types=[
        pltpu.VMEM((NCHUNK, CHUNK), jnp.int32),
        pltpu.VMEM((NCHUNK, CHUNK), jnp.int32),
        pltpu.VMEM((2, CHUNK, 16), jnp.float32),
        pltpu.VMEM((2, CHUNK, 16), jnp.float32),
        pltpu.VMEM((2, CHUNK, OUT_), jnp.float32),
        pltpu.VMEM((2, CHUNK, 16), jnp.float32),
        pltpu.VMEM((2, CHUNK, OUT_), jnp.float32),
        pltpu.VMEM_SHARED((NPAD, OUT_), jnp.float32),
        pltpu.VMEM_SHARED((NPAD, 16), jnp.float32),
        pltpu.SemaphoreType.DMA,
        pltpu.SemaphoreType.DMA,
    ],
    compiler_params=_sc_params,
)(_edge2_body)


def _ke_body(a0_ref, a1_ref, s0_ref, s1_ref, b2_ref, out_ref):
    acc = a0_ref[...] + a1_ref[...]
    ssum = s0_ref[...][:, 0:1] + s1_ref[...][:, 0:1]
    out_ref[...] = acc / (ssum + 1e-16) + b2_ref[...]


def _run_ke(a0, a1, s0, s1, b2row):
    blk = lambda r, c: pl.BlockSpec((r, c), lambda i: (i, 0))
    rep = lambda r, c: pl.BlockSpec((r, c), lambda i: (0, 0))
    return pl.pallas_call(
        _ke_body,
        grid=(5,),
        in_specs=[blk(2000, OUT_), blk(2000, OUT_), blk(2000, 16),
                  blk(2000, 16), rep(1, OUT_)],
        out_specs=blk(2000, OUT_),
        out_shape=jax.ShapeDtypeStruct((N_, OUT_), jnp.float32),
    )(a0, a1, s0, s1, b2row)


def kernel(x, edge_index, W1, a_src1, a_dst1, b1, W2, a_src2, a_dst2, b2):
    E = edge_index.shape[1]
    npad_e = EPAD - E - N_
    loop = jnp.arange(N_, dtype=jnp.int32)
    padidx = jnp.full((npad_e,), N_, dtype=jnp.int32)
    src = jnp.concatenate([edge_index[0], loop, padidx]).reshape(
        EPAD // CHUNK, CHUNK)
    dst = jnp.concatenate([edge_index[1], loop, padidx]).reshape(
        EPAD // CHUNK, CHUNK)

    xp = jnp.pad(x, ((0, NPAD - N_), (0, 0)))
    rows = jnp.arange(HC1)
    msrc = jnp.zeros((HC1, H1), jnp.float32).at[rows, rows // 8].set(
        a_src1.reshape(HC1))
    mdst = jnp.zeros((HC1, H1), jnp.float32).at[rows, rows // 8].set(
        a_dst1.reshape(HC1))
    erep = jnp.zeros((H1, HC1), jnp.float32).at[rows // 8, rows].set(1.0)

    h1, a1a, a1b = _run_ka(xp, W1, msrc, mdst)
    acc1, s1 = _edge1(h1, a1a, a1b, src, dst)
    h2, a2a, a2b = _run_kc(acc1[:NPAD], acc1[NPAD:], s1[:NPAD], s1[NPAD:],
                           b1.reshape(1, HC1), W2, erep,
                           a_src2.reshape(OUT_, 1), a_dst2.reshape(OUT_, 1))
    acc2, s2 = _edge2(h2, a2a, a2b, src, dst)
    return _run_ke(acc2[:N_], acc2[NPAD:NPAD + N_],
                   s2[:N_], s2[NPAD:NPAD + N_], b2.reshape(1, OUT_))


# speedup vs baseline: 1.0078x; 1.0025x over previous
import functools

import jax
import jax.numpy as jnp
from jax import lax
from jax.experimental import pallas as pl
from jax.experimental.pallas import tpu as pltpu
from jax.experimental.pallas import tpu_sc as plsc

N_ = 10000
D_ = 128
H1 = 8
HC1 = 64
OUT_ = 16
NPAD = 10112
NCORE = 2
NTILE = 16
CHUNK = 128
NCHUNK = 82
EPAD = NCORE * NTILE * NCHUNK * CHUNK
ROWS_PT = NPAD // NTILE
RB = NPAD // 8

_mesh = plsc.VectorSubcoreMesh(core_axis_name="c", subcore_axis_name="s")
_sc_params = pltpu.CompilerParams(use_tc_tiling_on_sc=False,
                                  needs_layout_passes=False)


def _dg16(v, idx):
    return lax.gather(
        v, idx[:, None],
        lax.GatherDimensionNumbers(
            offset_dims=(), collapsed_slice_dims=(0,), start_index_map=(0,)),
        slice_sizes=(1,),
        mode=lax.GatherScatterMode.PROMISE_IN_BOUNDS)


def _ka_body(x_ref, w1_ref, ms_ref, md_ref, h1_ref, a1a_ref, a1b_ref):
    h = jnp.dot(x_ref[...], w1_ref[...], preferred_element_type=jnp.float32)
    h1_ref[...] = h
    asrc = jnp.dot(h, ms_ref[...], preferred_element_type=jnp.float32)
    adst = jnp.dot(h, md_ref[...], preferred_element_type=jnp.float32)
    a1a_ref[...] = jnp.concatenate([asrc, adst], axis=1)
    a1b_ref[...] = jnp.concatenate([adst, asrc], axis=1)


def _run_ka(xp, W1, msrc, mdst):
    blk = lambda r, c: pl.BlockSpec((r, c), lambda i: (i, 0))
    rep = lambda r, c: pl.BlockSpec((r, c), lambda i: (0, 0))
    return pl.pallas_call(
        _ka_body,
        grid=(NPAD // RB,),
        in_specs=[blk(RB, D_), rep(D_, HC1), rep(HC1, H1), rep(HC1, H1)],
        out_specs=[blk(RB, HC1), blk(RB, 16), blk(RB, 16)],
        out_shape=[
            jax.ShapeDtypeStruct((NPAD, HC1), jnp.float32),
            jax.ShapeDtypeStruct((NPAD, 16), jnp.float32),
            jax.ShapeDtypeStruct((NPAD, 16), jnp.float32),
        ],
    )(xp, W1, msrc, mdst)


def _edge1_body(h1_hbm, a1a_hbm, a1b_hbm, src_hbm, dst_hbm,
                acc_out, s_out,
                src_v, dst_v, S_v, D_v, h_v, w_v, msg_v, acc_sh, s_sh,
                sem, sem2):
    c = lax.axis_index("c")
    s = lax.axis_index("s")
    wid = c * NTILE + s
    z16 = jnp.zeros((16,), jnp.float32)

    def zbody(i, carry):
        w_v[0, i] = z16
        for kk in range(4):
            msg_v[0, i, pl.ds(16 * kk, 16)] = z16
        return carry
    lax.fori_loop(0, CHUNK, zbody, 0)

    base = s * ROWS_PT
    for j in range(4):
        pltpu.sync_copy(msg_v.at[0], acc_sh.at[pl.ds(base + j * CHUNK, CHUNK)])
        pltpu.sync_copy(w_v.at[0], s_sh.at[pl.ds(base + j * CHUNK, CHUNK)])
    tail = ROWS_PT - 4 * CHUNK
    pltpu.sync_copy(msg_v.at[0].at[pl.ds(0, tail)],
                    acc_sh.at[pl.ds(base + 4 * CHUNK, tail)])
    pltpu.sync_copy(w_v.at[0].at[pl.ds(0, tail)],
                    s_sh.at[pl.ds(base + 4 * CHUNK, tail)])
    plsc.subcore_barrier()

    iota16 = lax.iota(jnp.int32, 16)
    hi8 = lax.shift_right_logical(iota16, 3)
    pats = [2 * kk + hi8 for kk in range(4)]

    pltpu.sync_copy(src_hbm.at[pl.ds(wid * NCHUNK, NCHUNK)], src_v)
    pltpu.sync_copy(dst_hbm.at[pl.ds(wid * NCHUNK, NCHUNK)], dst_v)

    def gathers(k, b):
        return [
            pltpu.make_async_copy(a1a_hbm.at[src_v.at[k]], S_v.at[b], sem),
            pltpu.make_async_copy(a1b_hbm.at[dst_v.at[k]], D_v.at[b], sem),
            pltpu.make_async_copy(h1_hbm.at[src_v.at[k]], h_v.at[b], sem),
        ]

    def scatters(k, b):
        return [
            pltpu.make_async_copy(w_v.at[b], s_sh.at[dst_v.at[k]], sem2),
            pltpu.make_async_copy(msg_v.at[b], acc_sh.at[dst_v.at[k]], sem2),
        ]

    for b in range(2):
        for g in gathers(b, b):
            g.start()

    def cbody(j, carry):
        for b in range(2):
            k = 2 * j + b
            for g in gathers(k, b):
                g.wait()

            @pl.when(k >= 2)
            def _():
                for sc in scatters(k - 2, b):
                    sc.wait()

            @plsc.parallel_loop(0, CHUNK, 1, unroll=4)
            def _(i):
                t = S_v[b, i] + D_v[b, i]
                w = jnp.exp(jnp.maximum(t, t * 0.2))
                w_v[b, i] = w
                for kk in range(4):
                    wexp = _dg16(w, pats[kk])
                    msg_v[b, i, pl.ds(16 * kk, 16)] = (
                        h_v[b, i, pl.ds(16 * kk, 16)] * wexp)

            pltpu.async_copy(w_v.at[b], s_sh.at[dst_v.at[k]], sem2, add=True)
            pltpu.async_copy(msg_v.at[b], acc_sh.at[dst_v.at[k]], sem2,
                             add=True)

            @pl.when(k + 2 < NCHUNK)
            def _():
                for g in gathers(k + 2, b):
                    g.start()
        return carry
    lax.fori_loop(0, NCHUNK // 2, cbody, 0)

    for b in range(2):
        for sc in scatters(NCHUNK - 2 + b, b):
            sc.wait()

    plsc.subcore_barrier()
    pltpu.sync_copy(acc_sh.at[pl.ds(base, ROWS_PT)],
                    acc_out.at[pl.ds(c * NPAD + base, ROWS_PT)])
    pltpu.sync_copy(s_sh.at[pl.ds(base, ROWS_PT)],
                    s_out.at[pl.ds(c * NPAD + base, ROWS_PT)])


_edge1 = functools.partial(
    pl.kernel,
    out_type=[
        jax.ShapeDtypeStruct((NCORE * NPAD, HC1), jnp.float32),
        jax.ShapeDtypeStruct((NCORE * NPAD, 16), jnp.float32),
    ],
    mesh=_mesh,
    scratch_types=[
        pltpu.VMEM((NCHUNK, CHUNK), jnp.int32),
        pltpu.VMEM((NCHUNK, CHUNK), jnp.int32),
        pltpu.VMEM((2, CHUNK, 16), jnp.float32),
        pltpu.VMEM((2, CHUNK, 16), jnp.float32),
        pltpu.VMEM((2, CHUNK, HC1), jnp.float32),
        pltpu.VMEM((2, CHUNK, 16), jnp.float32),
        pltpu.VMEM((2, CHUNK, HC1), jnp.float32),
        pltpu.VMEM_SHARED((NPAD, HC1), jnp.float32),
        pltpu.VMEM_SHARED((NPAD, 16), jnp.float32),
        pltpu.SemaphoreType.DMA,
        pltpu.SemaphoreType.DMA,
    ],
    compiler_params=_sc_params,
)(_edge1_body)


def _kc_body(a0_ref, a1_ref, s0_ref, s1_ref, b1_ref, w2_ref, erep_ref,
             as2_ref, ad2_ref, h2_ref, a2a_ref, a2b_ref):
    acc = a0_ref[...] + a1_ref[...]
    ssum = s0_ref[...][:, :H1] + s1_ref[...][:, :H1]
    inv = 1.0 / (ssum + 1e-16)
    inv_rep = jnp.dot(inv, erep_ref[...], preferred_element_type=jnp.float32)
    o1 = acc * inv_rep + b1_ref[...]
    hh = jnp.where(o1 > 0, o1, jnp.exp(o1) - 1.0)
    h2 = jnp.dot(hh, w2_ref[...], preferred_element_type=jnp.float32)
    h2_ref[...] = h2
    es = jnp.dot(h2, as2_ref[...], preferred_element_type=jnp.float32)
    ed = jnp.dot(h2, ad2_ref[...], preferred_element_type=jnp.float32)
    a2a_ref[...] = jnp.broadcast_to(es, (es.shape[0], 16))
    a2b_ref[...] = jnp.broadcast_to(ed, (ed.shape[0], 16))


def _run_kc(a0, a1, s0, s1, b1row, W2, erep, as2, ad2):
    blk = lambda r, c: pl.BlockSpec((r, c), lambda i: (i, 0))
    rep = lambda r, c: pl.BlockSpec((r, c), lambda i: (0, 0))
    return pl.pallas_call(
        _kc_body,
        grid=(NPAD // RB,),
        in_specs=[blk(RB, HC1), blk(RB, HC1), blk(RB, 16), blk(RB, 16),
                  rep(1, HC1), rep(HC1, OUT_), rep(H1, HC1),
                  rep(OUT_, 1), rep(OUT_, 1)],
        out_specs=[blk(RB, OUT_), blk(RB, 16), blk(RB, 16)],
        out_shape=[
            jax.ShapeDtypeStruct((NPAD, OUT_), jnp.float32),
            jax.ShapeDtypeStruct((NPAD, 16), jnp.float32),
            jax.ShapeDtypeStruct((NPAD, 16), jnp.float32),
        ],
    )(a0, a1, s0, s1, b1row, W2, erep, as2, ad2)


def _edge2_body(h2_hbm, a2a_hbm, a2b_hbm, src_hbm, dst_hbm,
                acc_out, s_out,
                src_v, dst_v, S_v, D_v, h_v, w_v, msg_v, acc_sh, s_sh,
                sem, sem2):
    c = lax.axis_index("c")
    s = lax.axis_index("s")
    wid = c * NTILE + s
    z16 = jnp.zeros((16,), jnp.float32)

    def zbody(i, carry):
        w_v[0, i] = z16
        msg_v[0, i] = z16
        return carry
    lax.fori_loop(0, CHUNK, zbody, 0)

    base = s * ROWS_PT
    for j in range(4):
        pltpu.sync_copy(msg_v.at[0], acc_sh.at[pl.ds(base + j * CHUNK, CHUNK)])
        pltpu.sync_copy(w_v.at[0], s_sh.at[pl.ds(base + j * CHUNK, CHUNK)])
    tail = ROWS_PT - 4 * CHUNK
    pltpu.sync_copy(msg_v.at[0].at[pl.ds(0, tail)],
                    acc_sh.at[pl.ds(base + 4 * CHUNK, tail)])
    pltpu.sync_copy(w_v.at[0].at[pl.ds(0, tail)],
                    s_sh.at[pl.ds(base + 4 * CHUNK, tail)])
    plsc.subcore_barrier()

    pltpu.sync_copy(src_hbm.at[pl.ds(wid * NCHUNK, NCHUNK)], src_v)
    pltpu.sync_copy(dst_hbm.at[pl.ds(wid * NCHUNK, NCHUNK)], dst_v)

    def gathers(k, b):
        return [
            pltpu.make_async_copy(a2a_hbm.at[src_v.at[k]], S_v.at[b], sem),
            pltpu.make_async_copy(a2b_hbm.at[dst_v.at[k]], D_v.at[b], sem),
            pltpu.make_async_copy(h2_hbm.at[src_v.at[k]], h_v.at[b], sem),
        ]

    def scatters(k, b):
        return [
            pltpu.make_async_copy(w_v.at[b], s_sh.at[dst_v.at[k]], sem2),
            pltpu.make_async_copy(msg_v.at[b], acc_sh.at[dst_v.at[k]], sem2),
        ]

    for b in range(2):
        for g in gathers(b, b):
            g.start()

    def cbody(j, carry):
        for b in range(2):
            k = 2 * j + b
            for g in gathers(k, b):
                g.wait()

            @pl.when(k >= 2)
            def _():
                for sc in scatters(k - 2, b):
                    sc.wait()

            @plsc.parallel_loop(0, CHUNK, 1, unroll=8)
            def _(i):
                t = S_v[b, i] + D_v[b, i]
                w = jnp.exp(jnp.maximum(t, t * 0.2))
                w_v[b, i] = w
                msg_v[b, i] = h_v[b, i] * w

            pltpu.async_copy(w_v.at[b], s_sh.at[dst_v.at[k]], sem2, add=True)
            pltpu.async_copy(msg_v.at[b], acc_sh.at[dst_v.at[k]], sem2,
                             add=True)

            @pl.when(k + 2 < NCHUNK)
            def _():
                for g in gathers(k + 2, b):
                    g.start()
        return carry
    lax.fori_loop(0, NCHUNK // 2, cbody, 0)

    for b in range(2):
        for sc in scatters(NCHUNK - 2 + b, b):
            sc.wait()

    plsc.subcore_barrier()
    pltpu.sync_copy(acc_sh.at[pl.ds(base, ROWS_PT)],
                    acc_out.at[pl.ds(c * NPAD + base, ROWS_PT)])
    pltpu.sync_copy(s_sh.at[pl.ds(base, ROWS_PT)],
                    s_out.at[pl.ds(c * NPAD + base, ROWS_PT)])


_edge2 = functools.partial(
    pl.kernel,
    out_type=[
        jax.ShapeDtypeStruct((NCORE * NPAD, OUT_), jnp.float32),
        jax.ShapeDtypeStruct((NCORE * NPAD, 16), jnp.float32),
    ],
    mesh=_mesh,
    scratch_types=[
        pltpu.VMEM((NCHUNK, CHUNK), jnp.int32),
        pltpu.VMEM((NCHUNK, CHUNK), jnp.int32),
        pltpu.VMEM((2, CHUNK, 16), jnp.float32),
        pltpu.VMEM((2, CHUNK, 16), jnp.float32),
        pltpu.VMEM((2, CHUNK, OUT_), jnp.float32),
        pltpu.VMEM((2, CHUNK, 16), jnp.float32),
        pltpu.VMEM((2, CHUNK, OUT_), jnp.float32),
        pltpu.VMEM_SHARED((NPAD, OUT_), jnp.float32),
        pltpu.VMEM_SHARED((NPAD, 16), jnp.float32),
        pltpu.SemaphoreType.DMA,
        pltpu.SemaphoreType.DMA,
    ],
    compiler_params=_sc_params,
)(_edge2_body)


def _ke_body(a0_ref, a1_ref, s0_ref, s1_ref, b2_ref, out_ref):
    acc = a0_ref[...] + a1_ref[...]
    ssum = s0_ref[...][:, 0:1] + s1_ref[...][:, 0:1]
    out_ref[...] = acc / (ssum + 1e-16) + b2_ref[...]


def _run_ke(a0, a1, s0, s1, b2row):
    blk = lambda r, c: pl.BlockSpec((r, c), lambda i: (i, 0))
    rep = lambda r, c: pl.BlockSpec((r, c), lambda i: (0, 0))
    return pl.pallas_call(
        _ke_body,
        grid=(5,),
        in_specs=[blk(2000, OUT_), blk(2000, OUT_), blk(2000, 16),
                  blk(2000, 16), rep(1, OUT_)],
        out_specs=blk(2000, OUT_),
        out_shape=jax.ShapeDtypeStruct((N_, OUT_), jnp.float32),
    )(a0, a1, s0, s1, b2row)


def kernel(x, edge_index, W1, a_src1, a_dst1, b1, W2, a_src2, a_dst2, b2):
    E = edge_index.shape[1]
    npad_e = EPAD - E - N_
    loop = jnp.arange(N_, dtype=jnp.int32)
    padidx = jnp.full((npad_e,), N_, dtype=jnp.int32)
    def _chunked(a):
        return a.reshape(NCHUNK, NCORE * NTILE, CHUNK).transpose(
            1, 0, 2).reshape(EPAD // CHUNK, CHUNK)
    src = _chunked(jnp.concatenate([edge_index[0], loop, padidx]))
    dst = _chunked(jnp.concatenate([edge_index[1], loop, padidx]))

    xp = jnp.pad(x, ((0, NPAD - N_), (0, 0)))
    rows = jnp.arange(HC1)
    msrc = jnp.zeros((HC1, H1), jnp.float32).at[rows, rows // 8].set(
        a_src1.reshape(HC1))
    mdst = jnp.zeros((HC1, H1), jnp.float32).at[rows, rows // 8].set(
        a_dst1.reshape(HC1))
    erep = jnp.zeros((H1, HC1), jnp.float32).at[rows // 8, rows].set(1.0)

    h1, a1a, a1b = _run_ka(xp, W1, msrc, mdst)
    acc1, s1 = _edge1(h1, a1a, a1b, src, dst)
    h2, a2a, a2b = _run_kc(acc1[:NPAD], acc1[NPAD:], s1[:NPAD], s1[NPAD:],
                           b1.reshape(1, HC1), W2, erep,
                           a_src2.reshape(OUT_, 1), a_dst2.reshape(OUT_, 1))
    acc2, s2 = _edge2(h2, a2a, a2b, src, dst)
    return _run_ke(acc2[:N_], acc2[NPAD:NPAD + N_],
                   s2[:N_], s2[NPAD:NPAD + N_], b2.reshape(1, OUT_))
